# baseline (speedup 1.0000x reference)
import functools

import jax
import jax.numpy as jnp
from jax import lax
from jax.experimental import pallas as pl
from jax.experimental.pallas import tpu as pltpu
from jax.experimental.pallas import tpu_sc as plsc

EMBED = 64
BATCH = 16384

NC = 2
NS = 16
NW = NC * NS
B_PER_W = BATCH // NW
CHUNK = 128
NCHUNK = B_PER_W // CHUNK

_mesh = plsc.VectorSubcoreMesh(core_axis_name="c", subcore_axis_name="s")


@functools.partial(
    pl.kernel,
    mesh=_mesh,
    compiler_params=pltpu.CompilerParams(use_tc_tiling_on_sc=False),
    out_type=[
        jax.ShapeDtypeStruct((BATCH, EMBED), jnp.float32),
        jax.ShapeDtypeStruct((BATCH, EMBED), jnp.float32),
        jax.ShapeDtypeStruct((BATCH, EMBED), jnp.float32),
    ],
    scratch_types=[
        pltpu.VMEM((NCHUNK, CHUNK), jnp.int32),
        pltpu.VMEM((NCHUNK, CHUNK), jnp.int32),
        pltpu.VMEM((NCHUNK, CHUNK), jnp.int32),
        pltpu.VMEM((B_PER_W, EMBED), jnp.float32),
        pltpu.VMEM((B_PER_W, EMBED), jnp.float32),
        pltpu.VMEM((B_PER_W, EMBED), jnp.float32),
        pltpu.SemaphoreType.DMA,
        pltpu.SemaphoreType.DMA,
    ],
)
def _gather3(users_hbm, pos_hbm, neg_hbm, uemb_hbm, iemb_hbm,
             out_u, out_p, out_n,
             idx_u, idx_p, idx_n, rows_u, rows_p, rows_n, sem_g, sem_s):
    wid = lax.axis_index("s") * NC + lax.axis_index("c")
    base = wid * B_PER_W

    pltpu.sync_copy(users_hbm.at[wid], idx_u)
    pltpu.sync_copy(pos_hbm.at[wid], idx_p)
    pltpu.sync_copy(neg_hbm.at[wid], idx_n)

    gathers = []
    for j in range(NCHUNK):
        sl = pl.ds(j * CHUNK, CHUNK)
        gathers.append(pltpu.async_copy(uemb_hbm.at[idx_u.at[j]], rows_u.at[sl], sem_g))
        gathers.append(pltpu.async_copy(iemb_hbm.at[idx_p.at[j]], rows_p.at[sl], sem_g))
        gathers.append(pltpu.async_copy(iemb_hbm.at[idx_n.at[j]], rows_n.at[sl], sem_g))
    for g in gathers:
        g.wait()

    out_sl = pl.ds(base, B_PER_W)
    stores = [
        pltpu.async_copy(rows_u, out_u.at[out_sl], sem_s),
        pltpu.async_copy(rows_p, out_p.at[out_sl], sem_s),
        pltpu.async_copy(rows_n, out_n.at[out_sl], sem_s),
    ]
    for s in stores:
        s.wait()


def kernel(users, pos_items, neg_items, _, user_emb, item_emb):
    u = users.astype(jnp.int32).reshape(NW, NCHUNK, CHUNK)
    p = pos_items.astype(jnp.int32).reshape(NW, NCHUNK, CHUNK)
    n = neg_items.astype(jnp.int32).reshape(NW, NCHUNK, CHUNK)
    out_u, out_p, out_n = _gather3(u, p, n, user_emb, item_emb)
    return out_u, out_p, out_n, _
